# Initial kernel scaffold; baseline (speedup 1.0000x reference)
import jax
import jax.numpy as jnp
from jax.experimental import pallas as pl


def kernel(ind, weight):
    raise NotImplementedError("write your pallas kernel here")


# speedup vs baseline: 1.6822x; 1.6822x over previous
import functools

import jax
import jax.numpy as jnp
from jax import lax
from jax.experimental import pallas as pl
from jax.experimental.pallas import tpu as pltpu
from jax.experimental.pallas import tpu_sc as plsc

VOCAB = 1000000
DIM = 64
B_TOK = 16384
SEQ = 50
NB = B_TOK * SEQ

NC = 2
NS = 16
NW = NC * NS

B_PER_W = NB // NW
CH = 128
NCHUNK = B_PER_W // CH

_mesh = plsc.VectorSubcoreMesh(core_axis_name="c", subcore_axis_name="s")


@functools.partial(
    pl.kernel,
    mesh=_mesh,
    out_type=jax.ShapeDtypeStruct((NB, DIM), jnp.float32),
    scratch_types=[
        pltpu.VMEM((NCHUNK, CH), jnp.int32),
        pltpu.VMEM((CH, DIM), jnp.float32),
        pltpu.SemaphoreType.DMA,
    ],
    compiler_params=pltpu.CompilerParams(use_tc_tiling_on_sc=False),
)
def _emb_lookup(ind_hbm, weight_hbm, out_hbm, idx_v, rows_v, sem):
    wid = lax.axis_index("s") * NC + lax.axis_index("c")
    pltpu.sync_copy(ind_hbm.at[wid], idx_v)
    row0 = wid * B_PER_W

    def body(g, carry):
        pltpu.async_copy(weight_hbm.at[idx_v.at[g]], rows_v, sem).wait()
        pltpu.sync_copy(rows_v, out_hbm.at[pl.ds(row0 + g * CH, CH)])
        return carry

    lax.fori_loop(0, NCHUNK, body, 0)


def kernel(ind, weight):
    ind_w = ind.reshape(NW, NCHUNK, CH)
    out = _emb_lookup(ind_w, weight)
    return out.reshape(B_TOK, SEQ, DIM)


# speedup vs baseline: 1.8723x; 1.1130x over previous
import functools

import jax
import jax.numpy as jnp
from jax import lax
from jax.experimental import pallas as pl
from jax.experimental.pallas import tpu as pltpu
from jax.experimental.pallas import tpu_sc as plsc

VOCAB = 1000000
DIM = 64
B_TOK = 16384
SEQ = 50
NB = B_TOK * SEQ

NC = 2
NS = 16
NW = NC * NS

B_PER_W = NB // NW
CH = 128
NCHUNK = B_PER_W // CH
NBUF = 8
NGRP = NCHUNK // NBUF

_mesh = plsc.VectorSubcoreMesh(core_axis_name="c", subcore_axis_name="s")


@functools.partial(
    pl.kernel,
    mesh=_mesh,
    out_type=jax.ShapeDtypeStruct((NB, DIM), jnp.float32),
    scratch_types=[
        pltpu.VMEM((NCHUNK, CH), jnp.int32),
        pltpu.VMEM((NBUF, CH, DIM), jnp.float32),
        pltpu.SemaphoreType.DMA((NBUF,)),
        pltpu.SemaphoreType.DMA((NBUF,)),
    ],
    compiler_params=pltpu.CompilerParams(use_tc_tiling_on_sc=False),
)
def _emb_lookup(ind_hbm, weight_hbm, out_hbm, idx_v, rows_v, gsem, wsem):
    wid = lax.axis_index("s") * NC + lax.axis_index("c")
    pltpu.sync_copy(ind_hbm.at[wid], idx_v)
    row0 = wid * B_PER_W

    def start_gather(chunk, b):
        pltpu.async_copy(weight_hbm.at[idx_v.at[chunk]], rows_v.at[b], gsem.at[b])

    def wait_gather(b):
        pltpu.make_async_copy(
            weight_hbm.at[idx_v.at[0]], rows_v.at[b], gsem.at[b]).wait()

    def wait_write(b):
        pltpu.make_async_copy(
            rows_v.at[b], out_hbm.at[pl.ds(row0, CH)], wsem.at[b]).wait()

    for b in range(NBUF):
        start_gather(b, b)

    def group(i, carry):
        for b in range(NBUF):
            wait_gather(b)
            pltpu.async_copy(
                rows_v.at[b], out_hbm.at[pl.ds(row0 + (i * NBUF + b) * CH, CH)],
                wsem.at[b])

        @pl.when(i < NGRP - 1)
        def _():
            for b in range(NBUF):
                wait_write(b)
                start_gather((i + 1) * NBUF + b, b)

        return carry

    lax.fori_loop(0, NGRP, group, 0)
    for b in range(NBUF):
        wait_write(b)


def kernel(ind, weight):
    ind_w = ind.reshape(NW, NCHUNK, CH)
    out = _emb_lookup(ind_w, weight)
    return out.reshape(B_TOK, SEQ, DIM)
